# baseline (speedup 1.0000x reference)
import functools

import jax
import jax.numpy as jnp
from jax import lax
from jax.experimental import pallas as pl
from jax.experimental.pallas import tpu as pltpu
from jax.experimental.pallas import tpu_sc as plsc

_NC, _NS = 2, 16
_NW = _NC * _NS
_B = 4096
_D = 64
_BPW = _B // _NW
_EPS = 1e-5
_L = 16


def _sc_gather_body(uid_hbm, iid_hbm, gu_tab, gi_tab, mu_tab, mi_tab,
                    gmf_out, mu_out, mi_out,
                    idx_u, idx_i, gu_v, gi_v, mu_v, mi_v, sem):
    wid = lax.axis_index("s") * _NC + lax.axis_index("c")
    base = wid * _BPW
    pltpu.sync_copy(uid_hbm.at[pl.ds(base, _BPW)], idx_u)
    pltpu.sync_copy(iid_hbm.at[pl.ds(base, _BPW)], idx_i)
    descs = []
    for g in range(_BPW // _L):
        vu = idx_u[pl.ds(g * _L, _L)] * _D
        vi = idx_i[pl.ds(g * _L, _L)] * _D
        for t in range(_L):
            r = g * _L + t
            id_u = pl.multiple_of(vu[t], _D)
            id_i = pl.multiple_of(vi[t], _D)
            dst = pl.ds(r * _D, _D)
            descs.append(pltpu.async_copy(
                gu_tab.at[pl.ds(id_u, _D)], gu_v.at[dst], sem))
            descs.append(pltpu.async_copy(
                gi_tab.at[pl.ds(id_i, _D)], gi_v.at[dst], sem))
            descs.append(pltpu.async_copy(
                mu_tab.at[pl.ds(id_u, _D)], mu_v.at[dst], sem))
            descs.append(pltpu.async_copy(
                mi_tab.at[pl.ds(id_i, _D)], mi_v.at[dst], sem))
    for d in descs:
        d.wait()
    out_sl = pl.ds(base * _D, _BPW * _D)
    pltpu.sync_copy(mu_v, mu_out.at[out_sl])
    pltpu.sync_copy(mi_v, mi_out.at[out_sl])

    def chunk16(i, carry):
        sl = pl.ds(i * _L, _L)
        gu_v[sl] = gu_v[sl] * gi_v[sl]
        return carry

    lax.fori_loop(0, _BPW * _D // _L, chunk16, 0)
    pltpu.sync_copy(gu_v, gmf_out.at[out_sl])


@jax.jit
def _sc_gather(user_ids, item_ids, gu_tab, gi_tab, mu_tab, mi_tab):
    mesh = plsc.VectorSubcoreMesh(core_axis_name="c", subcore_axis_name="s")
    f = pl.kernel(
        _sc_gather_body,
        out_type=(
            jax.ShapeDtypeStruct((_B * _D,), jnp.float32),
            jax.ShapeDtypeStruct((_B * _D,), jnp.float32),
            jax.ShapeDtypeStruct((_B * _D,), jnp.float32),
        ),
        mesh=mesh,
        scratch_types=[
            pltpu.VMEM((_BPW,), jnp.int32),
            pltpu.VMEM((_BPW,), jnp.int32),
            pltpu.VMEM((_BPW * _D,), jnp.float32),
            pltpu.VMEM((_BPW * _D,), jnp.float32),
            pltpu.VMEM((_BPW * _D,), jnp.float32),
            pltpu.VMEM((_BPW * _D,), jnp.float32),
            pltpu.SemaphoreType.DMA,
        ],
    )
    return f(user_ids, item_ids, gu_tab, gi_tab, mu_tab, mi_tab)


def _mlp_body(gmf_ref, mu_ref, mi_ref,
              w1u_ref, w1i_ref, w2_ref, w3_ref,
              b1_ref, s1_ref, be1_ref, b2_ref, s2_ref, be2_ref,
              b3_ref, s3_ref, be3_ref, wog_ref, woh_ref, bo_ref,
              out_ref):
    mu = mu_ref[...]
    mi = mi_ref[...]
    h = jnp.dot(mu, w1u_ref[...], preferred_element_type=jnp.float32)
    h = h + jnp.dot(mi, w1i_ref[...], preferred_element_type=jnp.float32)
    h = jnp.maximum(h + b1_ref[...], 0.0) * s1_ref[...] + be1_ref[...]
    h = jnp.dot(h, w2_ref[...], preferred_element_type=jnp.float32)
    h = jnp.maximum(h + b2_ref[...], 0.0) * s2_ref[...] + be2_ref[...]
    h = jnp.dot(h, w3_ref[...], preferred_element_type=jnp.float32)
    h = jnp.maximum(h + b3_ref[...], 0.0) * s3_ref[...] + be3_ref[...]
    logit = (jnp.sum(gmf_ref[...] * wog_ref[...], axis=-1)
             + jnp.sum(h * woh_ref[...], axis=-1) + bo_ref[0])
    out_ref[...] = jax.nn.sigmoid(logit)


@jax.jit
def _mlp_tower(gmf, mu, mi, w1u, w1i, w2, w3,
               b1, s1, be1, b2, s2, be2, b3, s3, be3, wog, woh, bo):
    nblk = 4
    rows = _B // nblk
    full = lambda i: (0, 0)
    batch2 = lambda shape: pl.BlockSpec((rows, shape), lambda i: (i, 0))
    return pl.pallas_call(
        _mlp_body,
        grid=(nblk,),
        in_specs=[
            batch2(_D), batch2(_D), batch2(_D),
            pl.BlockSpec((_D, 256), full), pl.BlockSpec((_D, 256), full),
            pl.BlockSpec((256, 128), full), pl.BlockSpec((128, _D), full),
            pl.BlockSpec((1, 256), full), pl.BlockSpec((1, 256), full),
            pl.BlockSpec((1, 256), full),
            pl.BlockSpec((1, 128), full), pl.BlockSpec((1, 128), full),
            pl.BlockSpec((1, 128), full),
            pl.BlockSpec((1, _D), full), pl.BlockSpec((1, _D), full),
            pl.BlockSpec((1, _D), full),
            pl.BlockSpec((1, _D), full), pl.BlockSpec((1, _D), full),
            pl.BlockSpec(memory_space=pltpu.SMEM),
        ],
        out_specs=pl.BlockSpec((rows,), lambda i: (i,)),
        out_shape=jax.ShapeDtypeStruct((_B,), jnp.float32),
    )(gmf, mu, mi, w1u, w1i, w2, w3,
      b1, s1, be1, b2, s2, be2, b3, s3, be3, wog, woh, bo)


def kernel(user_ids, item_ids, gmf_user_tab, gmf_item_tab, mlp_user_tab,
           mlp_item_tab, W1, b1, g1, be1, W2, b2, g2, be2, W3, b3, g3, be3,
           Wo, bo):
    user_ids = user_ids.astype(jnp.int32)
    item_ids = item_ids.astype(jnp.int32)
    flat = lambda t: t.reshape(-1)
    gmf, mu, mi = _sc_gather(user_ids, item_ids,
                             flat(gmf_user_tab), flat(gmf_item_tab),
                             flat(mlp_user_tab), flat(mlp_item_tab))
    gmf = gmf.reshape(_B, _D)
    mu = mu.reshape(_B, _D)
    mi = mi.reshape(_B, _D)
    inv = 1.0 / jnp.sqrt(1.0 + _EPS)
    w1u = W1[:, :_D].T
    w1i = W1[:, _D:].T
    r2 = lambda v: v.reshape(1, -1)
    return _mlp_tower(
        gmf, mu, mi, w1u, w1i, W2.T, W3.T,
        r2(b1), r2(inv * g1), r2(be1),
        r2(b2), r2(inv * g2), r2(be2),
        r2(b3), r2(inv * g3), r2(be3),
        r2(Wo[0, :_D]), r2(Wo[0, _D:]), bo)


# speedup vs baseline: 1.1790x; 1.1790x over previous
import functools

import jax
import jax.numpy as jnp
from jax import lax
from jax.experimental import pallas as pl
from jax.experimental.pallas import tpu as pltpu
from jax.experimental.pallas import tpu_sc as plsc

_NC, _NS = 2, 16
_NW = _NC * _NS
_B = 4096
_D = 64
_D2 = 2 * _D
_BPW = _B // _NW
_EPS = 1e-5


def _sc_gather_body(uid_hbm, iid_hbm, cat_u, cat_i,
                    du_out, di_out,
                    idx_u, idx_i, du_v, di_v, sem):
    wid = lax.axis_index("s") * _NC + lax.axis_index("c")
    base = wid * _BPW
    pltpu.sync_copy(uid_hbm.at[pl.ds(base, _BPW)], idx_u)
    pltpu.sync_copy(iid_hbm.at[pl.ds(base, _BPW)], idx_i)
    c1 = pltpu.async_copy(cat_u.at[idx_u], du_v, sem)
    c2 = pltpu.async_copy(cat_i.at[idx_i], di_v, sem)
    c1.wait()
    c2.wait()
    pltpu.sync_copy(du_v, du_out.at[pl.ds(base, _BPW)])
    pltpu.sync_copy(di_v, di_out.at[pl.ds(base, _BPW)])


@jax.jit
def _sc_gather(user_ids, item_ids, cat_u, cat_i):
    mesh = plsc.VectorSubcoreMesh(core_axis_name="c", subcore_axis_name="s")
    f = pl.kernel(
        _sc_gather_body,
        out_type=(
            jax.ShapeDtypeStruct((_B, _D2), jnp.float32),
            jax.ShapeDtypeStruct((_B, _D2), jnp.float32),
        ),
        mesh=mesh,
        scratch_types=[
            pltpu.VMEM((_BPW,), jnp.int32),
            pltpu.VMEM((_BPW,), jnp.int32),
            pltpu.VMEM((_BPW, _D2), jnp.float32),
            pltpu.VMEM((_BPW, _D2), jnp.float32),
            pltpu.SemaphoreType.DMA,
        ],
    )
    return f(user_ids, item_ids, cat_u, cat_i)


def _mlp_body(du_ref, di_ref,
              w1u_ref, w1i_ref, w2_ref, w3_ref,
              b1_ref, s1_ref, be1_ref, b2_ref, s2_ref, be2_ref,
              b3_ref, s3_ref, be3_ref, wog_ref, woh_ref, bo_ref,
              out_ref):
    du = du_ref[...]
    di = di_ref[...]
    h = jnp.dot(du, w1u_ref[...], preferred_element_type=jnp.float32)
    h = h + jnp.dot(di, w1i_ref[...], preferred_element_type=jnp.float32)
    h = jnp.maximum(h + b1_ref[...], 0.0) * s1_ref[...] + be1_ref[...]
    h = jnp.dot(h, w2_ref[...], preferred_element_type=jnp.float32)
    h = jnp.maximum(h + b2_ref[...], 0.0) * s2_ref[...] + be2_ref[...]
    h = jnp.dot(h, w3_ref[...], preferred_element_type=jnp.float32)
    h = jnp.maximum(h + b3_ref[...], 0.0) * s3_ref[...] + be3_ref[...]
    logit = (jnp.sum((du * di) * wog_ref[...], axis=-1)
             + jnp.sum(h * woh_ref[...], axis=-1) + bo_ref[0])
    out_ref[...] = jax.nn.sigmoid(logit)


@jax.jit
def _mlp_tower(du, di, w1u, w1i, w2, w3,
               b1, s1, be1, b2, s2, be2, b3, s3, be3, wog, woh, bo):
    nblk = 4
    rows = _B // nblk
    full = lambda i: (0, 0)
    batch2 = pl.BlockSpec((rows, _D2), lambda i: (i, 0))
    return pl.pallas_call(
        _mlp_body,
        grid=(nblk,),
        in_specs=[
            batch2, batch2,
            pl.BlockSpec((_D2, 256), full), pl.BlockSpec((_D2, 256), full),
            pl.BlockSpec((256, 128), full), pl.BlockSpec((128, _D), full),
            pl.BlockSpec((1, 256), full), pl.BlockSpec((1, 256), full),
            pl.BlockSpec((1, 256), full),
            pl.BlockSpec((1, 128), full), pl.BlockSpec((1, 128), full),
            pl.BlockSpec((1, 128), full),
            pl.BlockSpec((1, _D), full), pl.BlockSpec((1, _D), full),
            pl.BlockSpec((1, _D), full),
            pl.BlockSpec((1, _D2), full), pl.BlockSpec((1, _D), full),
            pl.BlockSpec(memory_space=pltpu.SMEM),
        ],
        out_specs=pl.BlockSpec((rows,), lambda i: (i,)),
        out_shape=jax.ShapeDtypeStruct((_B,), jnp.float32),
    )(du, di, w1u, w1i, w2, w3,
      b1, s1, be1, b2, s2, be2, b3, s3, be3, wog, woh, bo)


def kernel(user_ids, item_ids, gmf_user_tab, gmf_item_tab, mlp_user_tab,
           mlp_item_tab, W1, b1, g1, be1, W2, b2, g2, be2, W3, b3, g3, be3,
           Wo, bo):
    user_ids = user_ids.astype(jnp.int32)
    item_ids = item_ids.astype(jnp.int32)
    cat_u = jnp.concatenate([gmf_user_tab, mlp_user_tab], axis=1)
    cat_i = jnp.concatenate([gmf_item_tab, mlp_item_tab], axis=1)
    du, di = _sc_gather(user_ids, item_ids, cat_u, cat_i)
    inv = 1.0 / jnp.sqrt(1.0 + _EPS)
    z = jnp.zeros((_D, 256), jnp.float32)
    w1u = jnp.concatenate([z, W1[:, :_D].T], axis=0)
    w1i = jnp.concatenate([z, W1[:, _D:].T], axis=0)
    wog = jnp.concatenate([Wo[0, :_D], jnp.zeros((_D,), jnp.float32)])
    r2 = lambda v: v.reshape(1, -1)
    return _mlp_tower(
        du, di, w1u, w1i, W2.T, W3.T,
        r2(b1), r2(inv * g1), r2(be1),
        r2(b2), r2(inv * g2), r2(be2),
        r2(b3), r2(inv * g3), r2(be3),
        r2(wog), r2(Wo[0, _D:]), bo)


# speedup vs baseline: 3.1977x; 2.7122x over previous
import functools

import jax
import jax.numpy as jnp
from jax import lax
from jax.experimental import pallas as pl
from jax.experimental.pallas import tpu as pltpu
from jax.experimental.pallas import tpu_sc as plsc

_NC, _NS = 2, 16
_NW = _NC * _NS
_B = 4096
_D = 64
_N = 100000
_FPW = 4 * _D // _NW
_EPS = 1e-5
_L = 16


def _sc_gather_body(uid_hbm, iid_hbm, gu_t, mu_t, gi_t, mi_t,
                    out_hbm, idx_u, idx_i, row_v, out_v, sem):
    wid = lax.axis_index("s") * _NC + lax.axis_index("c")
    _FPT = _D // _NW
    fbase = wid * _FPT
    pltpu.sync_copy(uid_hbm, idx_u)
    pltpu.sync_copy(iid_hbm, idx_i)

    for t, (tab_ref, idx_v) in enumerate(
            ((gu_t, idx_u), (mu_t, idx_u), (gi_t, idx_i), (mi_t, idx_i))):
        def feat(f, carry, tab_ref=tab_ref, idx_v=idx_v, t=t):
            pltpu.sync_copy(tab_ref.at[fbase + f], row_v)

            def grp(g, c):
                iv = idx_v[pl.ds(g * _L, _L)]
                out_v[pl.ds(g * _L, _L)] = plsc.load_gather(row_v, [iv])
                return c

            lax.fori_loop(0, _B // _L, grp, 0)
            pltpu.sync_copy(out_v, out_hbm.at[t * _D + fbase + f])
            return carry

        lax.fori_loop(0, _FPT, feat, 0)


@jax.jit
def _sc_gather(user_ids, item_ids, gu_t, mu_t, gi_t, mi_t):
    mesh = plsc.VectorSubcoreMesh(core_axis_name="c", subcore_axis_name="s")
    f = pl.kernel(
        _sc_gather_body,
        out_type=jax.ShapeDtypeStruct((4 * _D, _B), jnp.float32),
        mesh=mesh,
        compiler_params=pltpu.CompilerParams(needs_layout_passes=False),
        scratch_types=[
            pltpu.VMEM((_B,), jnp.int32),
            pltpu.VMEM((_B,), jnp.int32),
            pltpu.VMEM((_N,), jnp.float32),
            pltpu.VMEM((_B,), jnp.float32),
            pltpu.SemaphoreType.DMA,
        ],
    )
    return f(user_ids, item_ids, gu_t, mu_t, gi_t, mi_t)


def _mlp_body(gu_ref, mu_ref, gi_ref, mi_ref,
              w1u_ref, w1i_ref, w2_ref, w3_ref,
              b1_ref, s1_ref, be1_ref, b2_ref, s2_ref, be2_ref,
              b3_ref, s3_ref, be3_ref, wog_ref, woh_ref, bo_ref,
              out_ref):
    cdim = (((0,), (0,)), ((), ()))
    h = lax.dot_general(w1u_ref[...], mu_ref[...], cdim,
                        preferred_element_type=jnp.float32)
    h = h + lax.dot_general(w1i_ref[...], mi_ref[...], cdim,
                            preferred_element_type=jnp.float32)
    h = jnp.maximum(h + b1_ref[...], 0.0) * s1_ref[...] + be1_ref[...]
    h = lax.dot_general(w2_ref[...], h, cdim,
                        preferred_element_type=jnp.float32)
    h = jnp.maximum(h + b2_ref[...], 0.0) * s2_ref[...] + be2_ref[...]
    h = lax.dot_general(w3_ref[...], h, cdim,
                        preferred_element_type=jnp.float32)
    h = jnp.maximum(h + b3_ref[...], 0.0) * s3_ref[...] + be3_ref[...]
    logit = (jnp.sum((gu_ref[...] * gi_ref[...]) * wog_ref[...], axis=0)
             + jnp.sum(h * woh_ref[...], axis=0) + bo_ref[0])
    out_ref[...] = jax.nn.sigmoid(logit)


@jax.jit
def _mlp_tower(bands, w1u, w1i, w2, w3,
               b1, s1, be1, b2, s2, be2, b3, s3, be3, wog, woh, bo):
    nblk = 4
    cols = _B // nblk
    full = lambda i: (0, 0)
    band = lambda r: pl.BlockSpec((_D, cols), lambda i, r=r: (r, i))
    return pl.pallas_call(
        _mlp_body,
        grid=(nblk,),
        in_specs=[
            band(0), band(1), band(2), band(3),
            pl.BlockSpec((_D, 256), full), pl.BlockSpec((_D, 256), full),
            pl.BlockSpec((256, 128), full), pl.BlockSpec((128, _D), full),
            pl.BlockSpec((256, 1), full), pl.BlockSpec((256, 1), full),
            pl.BlockSpec((256, 1), full),
            pl.BlockSpec((128, 1), full), pl.BlockSpec((128, 1), full),
            pl.BlockSpec((128, 1), full),
            pl.BlockSpec((_D, 1), full), pl.BlockSpec((_D, 1), full),
            pl.BlockSpec((_D, 1), full),
            pl.BlockSpec((_D, 1), full), pl.BlockSpec((_D, 1), full),
            pl.BlockSpec(memory_space=pltpu.SMEM),
        ],
        out_specs=pl.BlockSpec((cols,), lambda i: (i,)),
        out_shape=jax.ShapeDtypeStruct((_B,), jnp.float32),
    )(bands, bands, bands, bands, w1u, w1i, w2, w3,
      b1, s1, be1, b2, s2, be2, b3, s3, be3, wog, woh, bo)


def kernel(user_ids, item_ids, gmf_user_tab, gmf_item_tab, mlp_user_tab,
           mlp_item_tab, W1, b1, g1, be1, W2, b2, g2, be2, W3, b3, g3, be3,
           Wo, bo):
    user_ids = user_ids.astype(jnp.int32)
    item_ids = item_ids.astype(jnp.int32)
    bands = _sc_gather(user_ids, item_ids,
                       gmf_user_tab.T, mlp_user_tab.T,
                       gmf_item_tab.T, mlp_item_tab.T)
    inv = 1.0 / jnp.sqrt(1.0 + _EPS)
    w1u = W1[:, :_D].T
    w1i = W1[:, _D:].T
    col = lambda v: v.reshape(-1, 1)
    return _mlp_tower(
        bands, w1u, w1i, W2.T, W3.T,
        col(b1), col(inv * g1), col(be1),
        col(b2), col(inv * g2), col(be2),
        col(b3), col(inv * g3), col(be3),
        col(Wo[0, :_D]), col(Wo[0, _D:]), bo)


# speedup vs baseline: 3.7922x; 1.1859x over previous
import functools

import jax
import jax.numpy as jnp
from jax import lax
from jax.experimental import pallas as pl
from jax.experimental.pallas import tpu as pltpu
from jax.experimental.pallas import tpu_sc as plsc

_NC, _NS = 2, 16
_NW = _NC * _NS
_B = 4096
_D = 64
_N = 100000
_FPW = 4 * _D // _NW
_EPS = 1e-5
_L = 16


def _sc_gather_body(uid_hbm, iid_hbm, gu_t, mu_t, gi_t, mi_t,
                    out_hbm, idx_u, idx_i, row_v, out_v, sem):
    wid = lax.axis_index("s") * _NC + lax.axis_index("c")
    _FPT = _D // _NW
    fbase = wid * _FPT
    pltpu.sync_copy(uid_hbm, idx_u)
    pltpu.sync_copy(iid_hbm, idx_i)

    for t, (tab_ref, idx_v) in enumerate(
            ((gu_t, idx_u), (mu_t, idx_u), (gi_t, idx_i), (mi_t, idx_i))):
        def feat(f, carry, tab_ref=tab_ref, idx_v=idx_v, t=t):
            pltpu.sync_copy(tab_ref.at[fbase + f], row_v)

            @functools.partial(plsc.parallel_loop, 0, _B // _L, unroll=8)
            def _(g):
                iv = idx_v[pl.ds(g * _L, _L)]
                out_v[pl.ds(g * _L, _L)] = plsc.load_gather(row_v, [iv])
            pltpu.sync_copy(out_v, out_hbm.at[t * _D + fbase + f])
            return carry

        lax.fori_loop(0, _FPT, feat, 0)


@jax.jit
def _sc_gather(user_ids, item_ids, gu_t, mu_t, gi_t, mi_t):
    mesh = plsc.VectorSubcoreMesh(core_axis_name="c", subcore_axis_name="s")
    f = pl.kernel(
        _sc_gather_body,
        out_type=jax.ShapeDtypeStruct((4 * _D, _B), jnp.float32),
        mesh=mesh,
        compiler_params=pltpu.CompilerParams(needs_layout_passes=False),
        scratch_types=[
            pltpu.VMEM((_B,), jnp.int32),
            pltpu.VMEM((_B,), jnp.int32),
            pltpu.VMEM((_N,), jnp.float32),
            pltpu.VMEM((_B,), jnp.float32),
            pltpu.SemaphoreType.DMA,
        ],
    )
    return f(user_ids, item_ids, gu_t, mu_t, gi_t, mi_t)


def _mlp_body(gu_ref, mu_ref, gi_ref, mi_ref,
              w1_ref, w2_ref, w3_ref,
              b1_ref, s1_ref, be1_ref, b2_ref, s2_ref, be2_ref,
              b3_ref, s3_ref, be3_ref, wog_ref, woh_ref, bo_ref,
              out_ref):
    cdim = (((1,), (0,)), ((), ()))
    w1 = w1_ref[...]
    h = lax.dot_general(w1[:, :_D], mu_ref[...], cdim,
                        preferred_element_type=jnp.float32)
    h = h + lax.dot_general(w1[:, _D:], mi_ref[...], cdim,
                            preferred_element_type=jnp.float32)
    h = jnp.maximum(h + b1_ref[...], 0.0) * s1_ref[...] + be1_ref[...]
    h = lax.dot_general(w2_ref[...], h, cdim,
                        preferred_element_type=jnp.float32)
    h = jnp.maximum(h + b2_ref[...], 0.0) * s2_ref[...] + be2_ref[...]
    h = lax.dot_general(w3_ref[...], h, cdim,
                        preferred_element_type=jnp.float32)
    h = jnp.maximum(h + b3_ref[...], 0.0) * s3_ref[...] + be3_ref[...]
    logit = (jnp.sum((gu_ref[...] * gi_ref[...]) * wog_ref[...], axis=0)
             + jnp.sum(h * woh_ref[...], axis=0) + bo_ref[0])
    out_ref[...] = jax.nn.sigmoid(logit)


@jax.jit
def _mlp_tower(bands, w1, w2, w3,
               b1, s1, be1, b2, s2, be2, b3, s3, be3, wog, woh, bo):
    nblk = 4
    cols = _B // nblk
    full = lambda i: (0, 0)
    band = lambda r: pl.BlockSpec((_D, cols), lambda i, r=r: (r, i))
    return pl.pallas_call(
        _mlp_body,
        grid=(nblk,),
        in_specs=[
            band(0), band(1), band(2), band(3),
            pl.BlockSpec((256, 128), full),
            pl.BlockSpec((128, 256), full), pl.BlockSpec((_D, 128), full),
            pl.BlockSpec((256, 1), full), pl.BlockSpec((256, 1), full),
            pl.BlockSpec((256, 1), full),
            pl.BlockSpec((128, 1), full), pl.BlockSpec((128, 1), full),
            pl.BlockSpec((128, 1), full),
            pl.BlockSpec((_D, 1), full), pl.BlockSpec((_D, 1), full),
            pl.BlockSpec((_D, 1), full),
            pl.BlockSpec((_D, 1), full), pl.BlockSpec((_D, 1), full),
            pl.BlockSpec(memory_space=pltpu.SMEM),
        ],
        out_specs=pl.BlockSpec((cols,), lambda i: (i,)),
        out_shape=jax.ShapeDtypeStruct((_B,), jnp.float32),
    )(bands, bands, bands, bands, w1, w2, w3,
      b1, s1, be1, b2, s2, be2, b3, s3, be3, wog, woh, bo)


def kernel(user_ids, item_ids, gmf_user_tab, gmf_item_tab, mlp_user_tab,
           mlp_item_tab, W1, b1, g1, be1, W2, b2, g2, be2, W3, b3, g3, be3,
           Wo, bo):
    user_ids = user_ids.astype(jnp.int32)
    item_ids = item_ids.astype(jnp.int32)
    bands = _sc_gather(user_ids, item_ids,
                       gmf_user_tab.T, mlp_user_tab.T,
                       gmf_item_tab.T, mlp_item_tab.T)
    inv = 1.0 / jnp.sqrt(1.0 + _EPS)
    col = lambda v: v.reshape(-1, 1)
    return _mlp_tower(
        bands, W1, W2, W3,
        col(b1), col(inv * g1), col(be1),
        col(b2), col(inv * g2), col(be2),
        col(b3), col(inv * g3), col(be3),
        col(Wo[0, :_D]), col(Wo[0, _D:]), bo)
